# baseline (speedup 1.0000x reference)
import functools

import jax
import jax.numpy as jnp
from jax import lax
from jax.experimental import pallas as pl
from jax.experimental.pallas import tpu as pltpu
from jax.experimental.pallas import tpu_sc as plsc

C = 50
N = 16384
B = C * N
D = 32
NW = 32
PER_W = B // NW
CHUNK = 128
ITEMS_W = PER_W // CHUNK
NBUF = 8
ROUNDS = ITEMS_W // NBUF


def _sc_gather(idx_flat, table):
    mesh = plsc.VectorSubcoreMesh(core_axis_name="c", subcore_axis_name="s")

    @functools.partial(
        pl.kernel,
        mesh=mesh,
        out_type=jax.ShapeDtypeStruct((C, D // 8, N // 128, 8 * 128),
                                      jnp.float32),
        compiler_params=pltpu.CompilerParams(
            use_tc_tiling_on_sc=False, needs_layout_passes=False),
        scratch_types=[
            pltpu.VMEM((PER_W,), jnp.int32),
            *[pltpu.VMEM((CHUNK, D), jnp.float32) for _ in range(NBUF)],
            *[pltpu.VMEM((D * 128,), jnp.float32) for _ in range(NBUF)],
            *[pltpu.SemaphoreType.DMA for _ in range(NBUF)],
            *[pltpu.SemaphoreType.DMA for _ in range(NBUF)],
        ],
    )
    def k(idx_hbm, table_hbm, out_hbm, idx_v, *bufs_sems):
        rows = bufs_sems[:NBUF]
        rowsT = bufs_sems[NBUF:2 * NBUF]
        gsem = bufs_sems[2 * NBUF:3 * NBUF]
        ssem = bufs_sems[3 * NBUF:4 * NBUF]

        wid = lax.axis_index("s") * 2 + lax.axis_index("c")
        base = wid * ITEMS_W

        iota = lax.iota(jnp.int32, 16)

        pltpu.sync_copy(idx_hbm.at[pl.ds(base * CHUNK, PER_W)], idx_v)

        def gather_start(j, b):
            pltpu.async_copy(
                table_hbm.at[idx_v.at[pl.ds(j * CHUNK, CHUNK)]], rows[b],
                gsem[b])

        def transpose(b):
            def tbody(m, carry):
                ivec = iota + m * 16
                for d in range(D):
                    val = plsc.load_gather(
                        rows[b], [ivec, jnp.full((16,), d, jnp.int32)])
                    rowsT[b][pl.ds(d * 128 + m * 16, 16)] = val
                return carry

            lax.fori_loop(0, 8, tbody, 0)

        def store_start(j, b):
            t = base + j
            col = t >> 7
            tc = t & 127
            for tr in range(4):
                pltpu.async_copy(
                    rowsT[b].at[pl.ds(tr * 1024, 1024)],
                    out_hbm.at[col, tr, tc], ssem[b])

        def drain_gather(b):
            pltpu.make_async_copy(
                table_hbm.at[pl.ds(0, CHUNK)], rows[b], gsem[b]).wait()

        def drain_store(b):
            for tr in range(4):
                pltpu.make_async_copy(
                    rowsT[b].at[pl.ds(tr * 1024, 1024)],
                    out_hbm.at[0, 0, 0], ssem[b]).wait()

        for b in range(NBUF):
            gather_start(b, b)

        for b in range(NBUF):
            drain_gather(b)
            transpose(b)
            store_start(b, b)
            gather_start(b + NBUF, b)

        def body(r, carry):
            g = r * NBUF
            for b in range(NBUF):
                j = g + b
                drain_gather(b)
                drain_store(b)
                transpose(b)
                store_start(j, b)
                gather_start(j + NBUF, b)
            return carry

        lax.fori_loop(1, ROUNDS - 1, body, 0)

        for b in range(NBUF):
            j = ITEMS_W - NBUF + b
            drain_gather(b)
            drain_store(b)
            transpose(b)
            store_start(j, b)
        for b in range(NBUF):
            drain_store(b)

    return k(idx_flat, table)


def kernel(x, table):
    idx = x.T.reshape(-1).astype(jnp.int32)
    out5 = _sc_gather(idx, table)
    out = out5.reshape(C, D // 8, N // 128, 8, 128)
    out = jnp.transpose(out, (2, 4, 0, 1, 3))
    return out.reshape(N, C, D)


# speedup vs baseline: 1.3029x; 1.3029x over previous
import functools

import jax
import jax.numpy as jnp
from jax import lax
from jax.experimental import pallas as pl
from jax.experimental.pallas import tpu as pltpu
from jax.experimental.pallas import tpu_sc as plsc

C = 50
N = 16384
B = C * N
D = 32
NW = 32
PER_W = B // NW
CHUNK = 128
ITEMS_W = PER_W // CHUNK
NBUF = 8
ROUNDS = ITEMS_W // NBUF


def _sc_gather(idx_flat, table):
    mesh = plsc.VectorSubcoreMesh(core_axis_name="c", subcore_axis_name="s")

    @functools.partial(
        pl.kernel,
        mesh=mesh,
        out_type=jax.ShapeDtypeStruct((C, D // 8, N // 128, 8 * 128),
                                      jnp.float32),
        compiler_params=pltpu.CompilerParams(
            use_tc_tiling_on_sc=False, needs_layout_passes=False),
        scratch_types=[
            pltpu.VMEM((PER_W,), jnp.int32),
            *[pltpu.VMEM((CHUNK, D), jnp.float32) for _ in range(NBUF)],
            *[pltpu.VMEM((D * 128,), jnp.float32) for _ in range(NBUF)],
            *[pltpu.SemaphoreType.DMA for _ in range(NBUF)],
            *[pltpu.SemaphoreType.DMA for _ in range(NBUF)],
        ],
    )
    def k(idx_hbm, table_hbm, out_hbm, idx_v, *bufs_sems):
        rows = bufs_sems[:NBUF]
        rowsT = bufs_sems[NBUF:2 * NBUF]
        gsem = bufs_sems[2 * NBUF:3 * NBUF]
        ssem = bufs_sems[3 * NBUF:4 * NBUF]

        wid = lax.axis_index("s") * 2 + lax.axis_index("c")
        base = wid * ITEMS_W

        iota = lax.iota(jnp.int32, 16)
        dsplats = [jnp.full((16,), d, jnp.int32) for d in range(D)]

        pltpu.sync_copy(idx_hbm.at[pl.ds(base * CHUNK, PER_W)], idx_v)

        def gather_start(j, b):
            pltpu.async_copy(
                table_hbm.at[idx_v.at[pl.ds(j * CHUNK, CHUNK)]], rows[b],
                gsem[b])

        def transpose(b):
            def tbody(m, carry):
                ivec = iota + m * 16
                vals = [plsc.load_gather(rows[b], [ivec, dsplats[d]])
                        for d in range(D)]
                for d in range(D):
                    rowsT[b][pl.ds(d * 128 + m * 16, 16)] = vals[d]
                return carry

            lax.fori_loop(0, 8, tbody, 0)

        def store_start(j, b):
            t = base + j
            col = t >> 7
            tc = t & 127
            for tr in range(4):
                pltpu.async_copy(
                    rowsT[b].at[pl.ds(tr * 1024, 1024)],
                    out_hbm.at[col, tr, tc], ssem[b])

        def drain_gather(b):
            pltpu.make_async_copy(
                table_hbm.at[pl.ds(0, CHUNK)], rows[b], gsem[b]).wait()

        def drain_store(b):
            for tr in range(4):
                pltpu.make_async_copy(
                    rowsT[b].at[pl.ds(tr * 1024, 1024)],
                    out_hbm.at[0, 0, 0], ssem[b]).wait()

        for b in range(NBUF):
            gather_start(b, b)

        for b in range(NBUF):
            drain_gather(b)
            transpose(b)
            store_start(b, b)
            gather_start(b + NBUF, b)

        def body(r, carry):
            g = r * NBUF
            for b in range(NBUF):
                j = g + b
                drain_gather(b)
                drain_store(b)
                transpose(b)
                store_start(j, b)
                gather_start(j + NBUF, b)
            return carry

        lax.fori_loop(1, ROUNDS - 1, body, 0)

        for b in range(NBUF):
            j = ITEMS_W - NBUF + b
            drain_gather(b)
            drain_store(b)
            transpose(b)
            store_start(j, b)
        for b in range(NBUF):
            drain_store(b)

    return k(idx_flat, table)


def kernel(x, table):
    idx = x.T.reshape(-1).astype(jnp.int32)
    out5 = _sc_gather(idx, table)
    out = out5.reshape(C, D // 8, N // 128, 8, 128)
    out = jnp.transpose(out, (2, 4, 0, 1, 3))
    return out.reshape(N, C, D)


# speedup vs baseline: 1.3134x; 1.0081x over previous
import functools

import jax
import jax.numpy as jnp
from jax import lax
from jax.experimental import pallas as pl
from jax.experimental.pallas import tpu as pltpu
from jax.experimental.pallas import tpu_sc as plsc

C = 50
N = 16384
B = C * N
D = 32
NW = 32
PER_W = B // NW
GSZ = 512
GROUPS_W = PER_W // GSZ
NTR = D // 8


def _sc_gather(idx_flat, table):
    mesh = plsc.VectorSubcoreMesh(core_axis_name="c", subcore_axis_name="s")

    @functools.partial(
        pl.kernel,
        mesh=mesh,
        out_type=jax.ShapeDtypeStruct((C, NTR, N // 128, 8 * 128),
                                      jnp.float32),
        compiler_params=pltpu.CompilerParams(
            use_tc_tiling_on_sc=False, needs_layout_passes=False),
        scratch_types=[
            pltpu.VMEM((PER_W,), jnp.int32),
            *[pltpu.VMEM((GSZ, D), jnp.float32) for _ in range(2)],
            *[pltpu.VMEM((NTR, 4, 8 * 128), jnp.float32) for _ in range(2)],
            *[pltpu.SemaphoreType.DMA for _ in range(2)],
            *[pltpu.SemaphoreType.DMA for _ in range(2)],
        ],
    )
    def k(idx_hbm, table_hbm, out_hbm, idx_v, *bufs_sems):
        rows = bufs_sems[0:2]
        rowsT = bufs_sems[2:4]
        gsem = bufs_sems[4:6]
        ssem = bufs_sems[6:8]

        wid = lax.axis_index("s") * 2 + lax.axis_index("c")
        base = wid * GROUPS_W

        iota = lax.iota(jnp.int32, 16)
        dsplats = [jnp.full((16,), d, jnp.int32) for d in range(D)]

        pltpu.sync_copy(idx_hbm.at[pl.ds(base * GSZ, PER_W)], idx_v)

        def gather_start(g, p):
            pltpu.async_copy(
                table_hbm.at[idx_v.at[pl.ds((g - base) * GSZ, GSZ)]], rows[p],
                gsem[p])

        def transpose(p):
            def tbody(m, carry):
                for s in range(4):
                    ivec = iota + (s * 128 + m * 16)
                    for half in range(2):
                        vals = [
                            plsc.load_gather(rows[p], [ivec, dsplats[d]])
                            for d in range(half * 16, half * 16 + 16)
                        ]
                        for q, v in enumerate(vals):
                            d = half * 16 + q
                            rowsT[p][d >> 3, s,
                                     pl.ds((d & 7) * 128 + m * 16, 16)] = v
                return carry

            lax.fori_loop(0, 8, tbody, 0)

        def store_start(g, p):
            t = g * 4
            col = t >> 7
            tc = t & 127
            for tr in range(NTR):
                pltpu.async_copy(
                    rowsT[p].at[tr], out_hbm.at[col, tr, pl.ds(tc, 4)],
                    ssem[p])

        def drain_gather(p):
            pltpu.make_async_copy(
                table_hbm.at[pl.ds(0, GSZ)], rows[p], gsem[p]).wait()

        def drain_store(p):
            for tr in range(NTR):
                pltpu.make_async_copy(
                    rowsT[p].at[tr], out_hbm.at[0, 0, pl.ds(0, 4)],
                    ssem[p]).wait()

        def process(g, p, first):
            drain_gather(p)
            if not first:
                drain_store(p)
            transpose(p)
            store_start(g, p)

        gather_start(base, 0)
        gather_start(base + 1, 1)

        for p in range(2):
            process(base + p, p, True)
            gather_start(base + p + 2, p)

        def body(gg, carry):
            g = base + gg * 2
            for p in range(2):
                process(g + p, p, False)
                gather_start(g + p + 2, p)
            return carry

        lax.fori_loop(1, GROUPS_W // 2 - 1, body, 0)

        for p in range(2):
            process(base + GROUPS_W - 2 + p, p, False)
        for p in range(2):
            drain_store(p)

    return k(idx_flat, table)


def kernel(x, table):
    idx = x.T.reshape(-1).astype(jnp.int32)
    out5 = _sc_gather(idx, table)
    out = out5.reshape(C, NTR, N // 128, 8, 128)
    out = jnp.transpose(out, (2, 4, 0, 1, 3))
    return out.reshape(N, C, D)


# speedup vs baseline: 1.5748x; 1.1990x over previous
import functools

import jax
import jax.numpy as jnp
from jax import lax
from jax.experimental import pallas as pl
from jax.experimental.pallas import tpu as pltpu
from jax.experimental.pallas import tpu_sc as plsc

C = 50
N = 16384
B = C * N
D = 32
NW = 32
PER_W = B // NW
GSZ = 512
GROUPS_W = PER_W // GSZ
NTR = D // 8


def _sc_gather(idx_flat, table):
    mesh = plsc.VectorSubcoreMesh(core_axis_name="c", subcore_axis_name="s")

    @functools.partial(
        pl.kernel,
        mesh=mesh,
        out_type=jax.ShapeDtypeStruct((C, NTR, N // 128, 8, 128),
                                      jnp.float32),
        compiler_params=pltpu.CompilerParams(
            use_tc_tiling_on_sc=False, needs_layout_passes=False),
        scratch_types=[
            pltpu.VMEM((PER_W,), jnp.int32),
            *[pltpu.VMEM((GSZ, D), jnp.float32) for _ in range(2)],
            *[pltpu.VMEM((NTR, 4, 8, 133), jnp.float32) for _ in range(2)],
            *[pltpu.SemaphoreType.DMA for _ in range(2)],
            *[pltpu.SemaphoreType.DMA for _ in range(2)],
        ],
    )
    def k(idx_hbm, table_hbm, out_hbm, idx_v, *bufs_sems):
        rows = bufs_sems[0:2]
        rowsT = bufs_sems[2:4]
        gsem = bufs_sems[4:6]
        ssem = bufs_sems[6:8]

        wid = lax.axis_index("s") * 2 + lax.axis_index("c")
        base = wid * GROUPS_W

        iota = lax.iota(jnp.int32, 16)
        dd_vec = iota & 7
        tr_vecs = [(iota >> 3) + 2 * h for h in range(2)]
        s_vecs = [jnp.full((16,), s, jnp.int32) for s in range(4)]

        pltpu.sync_copy(idx_hbm.at[pl.ds(base * GSZ, PER_W)], idx_v)

        def gather_start(g, p):
            pltpu.async_copy(
                table_hbm.at[idx_v.at[pl.ds((g - base) * GSZ, GSZ)]], rows[p],
                gsem[p])

        def transpose(p):
            def tbody(m, carry):
                for s in range(4):
                    for u in range(4):
                        i = m * 4 + u
                        iv = jnp.full((16,), i, jnp.int32)
                        row = s * 128 + i
                        for h in range(2):
                            val = rows[p][row, pl.ds(h * 16, 16)]
                            plsc.store_scatter(
                                rowsT[p], [tr_vecs[h], s_vecs[s], dd_vec, iv],
                                val)
                return carry

            lax.fori_loop(0, 32, tbody, 0)

        def store_start(g, p):
            t = g * 4
            col = t >> 7
            tc = t & 127
            for tr in range(NTR):
                pltpu.async_copy(
                    rowsT[p].at[tr, :, :, pl.ds(0, 128)],
                    out_hbm.at[col, tr, pl.ds(tc, 4)], ssem[p])

        def drain_gather(p):
            pltpu.make_async_copy(
                table_hbm.at[pl.ds(0, GSZ)], rows[p], gsem[p]).wait()

        def drain_store(p):
            for tr in range(NTR):
                pltpu.make_async_copy(
                    rowsT[p].at[tr, :, :, pl.ds(0, 128)],
                    out_hbm.at[0, 0, pl.ds(0, 4)], ssem[p]).wait()

        def process(g, p, first):
            drain_gather(p)
            if not first:
                drain_store(p)
            transpose(p)
            store_start(g, p)

        gather_start(base, 0)
        gather_start(base + 1, 1)

        for p in range(2):
            process(base + p, p, True)
            gather_start(base + p + 2, p)

        def body(gg, carry):
            g = base + gg * 2
            for p in range(2):
                process(g + p, p, False)
                gather_start(g + p + 2, p)
            return carry

        lax.fori_loop(1, GROUPS_W // 2 - 1, body, 0)

        for p in range(2):
            process(base + GROUPS_W - 2 + p, p, False)
        for p in range(2):
            drain_store(p)

    return k(idx_flat, table)


def kernel(x, table):
    idx = x.T.reshape(-1).astype(jnp.int32)
    out5 = _sc_gather(idx, table)
    out = jnp.transpose(out5, (2, 4, 0, 1, 3))
    return out.reshape(N, C, D)
